# baseline (speedup 1.0000x reference)
import functools

import jax
import jax.numpy as jnp
from jax import lax
from jax.experimental import pallas as pl
from jax.experimental.pallas import tpu as pltpu
from jax.experimental.pallas import tpu_sc as plsc

HIDDEN = 768
INTER = 1536
E = 8
CAP = 256
T = 2048
EPS = 1e-06

NC = 2
NS = 16
L = 16
NW = NC * NS
TPW = T // NW


def _sc_mesh():
    return plsc.VectorSubcoreMesh(
        core_axis_name="c", subcore_axis_name="s", num_cores=NC, num_subcores=NS
    )


_SC_PARAMS = pltpu.CompilerParams(needs_layout_passes=False)


def _wid():
    return lax.axis_index("s") * NC + lax.axis_index("c")


def _router(x_flat, gate_norm_w, gate_w):
    var = jnp.mean(jnp.square(x_flat), axis=-1, keepdims=True)
    x_norm = x_flat * lax.rsqrt(var + EPS) * gate_norm_w
    v2 = jnp.mean(jnp.square(x_norm), axis=-1, keepdims=True)
    xr = x_norm * lax.rsqrt(v2 + 1e-08)
    s = 127.0 / jnp.clip(jnp.max(jnp.abs(xr), axis=-1, keepdims=True), 1e-05, None)
    xq = jnp.clip(jnp.round(xr * s), -128, 127) / s
    xq = xr + lax.stop_gradient(xq - xr)
    ws = 1.0 / jnp.clip(jnp.mean(jnp.abs(gate_w)), 1e-05, None)
    wq = jnp.clip(jnp.round(gate_w * ws), -1, 1) / ws
    wq = gate_w + lax.stop_gradient(wq - gate_w)
    logits = xq @ wq.T
    probs = jax.nn.softmax(logits, axis=-1)
    top_w, top_i = jax.lax.top_k(probs, 2)
    return top_w, top_i


def _build_compact_gather(interpret=False):
    @functools.partial(
        pl.kernel,
        out_type=(
            jax.ShapeDtypeStruct((E * CAP, HIDDEN), jnp.float32),
            jax.ShapeDtypeStruct((E, T), jnp.int32),
        ),
        mesh=_sc_mesh(),
        scratch_types=[
            pltpu.VMEM((2 * T,), jnp.int32),
            pltpu.VMEM((CAP,), jnp.int32),
            pltpu.VMEM((T,), jnp.int32),
            pltpu.VMEM((TPW, HIDDEN), jnp.float32),
            pltpu.SemaphoreType.DMA,
        ],
        compiler_params=_SC_PARAMS,
        interpret=interpret,
    )
    def compact_gather(ti_hbm, x_hbm, disp_out, pos_out,
                       ti_v, tok_v, pos_v, rows_v, sem):
        wid = _wid()
        e = wid // (NW // E)
        sub = wid % (NW // E)
        pltpu.sync_copy(ti_hbm, ti_v)

        def zero_body(i, carry):
            tok_v[pl.ds(i * L, L)] = jnp.zeros((L,), jnp.int32)
            return carry

        lax.fori_loop(0, CAP // L, zero_body, 0)

        def chunk(i, running):
            base = i * L
            tid = base + lax.iota(jnp.int32, L)
            i0 = plsc.load_gather(ti_v, [2 * tid])
            i1 = plsc.load_gather(ti_v, [2 * tid + 1])
            sel = (i0 == e) | (i1 == e)
            inc = sel.astype(jnp.int32)
            pos = running + plsc.cumsum(inc) - 1
            keep = sel & (pos < CAP)
            pos_v[pl.ds(base, L)] = jnp.where(keep, pos, -1)
            plsc.store_scatter(tok_v, [jnp.where(keep, pos, 0)], tid, mask=keep)
            return running + jnp.sum(inc)

        lax.fori_loop(0, T // L, chunk, 0)

        @pl.when(sub == 0)
        def _():
            pltpu.sync_copy(pos_v, pos_out.at[e])

        pltpu.async_copy(
            x_hbm.at[tok_v.at[pl.ds(sub * TPW, TPW)]], rows_v, sem
        ).wait()
        pltpu.sync_copy(rows_v, disp_out.at[pl.ds(wid * TPW, TPW)])

    return compact_gather


def _quant_act(x):
    v = jnp.mean(jnp.square(x), axis=-1, keepdims=True)
    xr = x * lax.rsqrt(v + 1e-08)
    s = 127.0 / jnp.clip(jnp.max(jnp.abs(xr), axis=-1, keepdims=True), 1e-05, None)
    q = jnp.clip(jnp.round(xr * s), -128, 127) / s
    return xr + (q - xr)


def _quant_w(w):
    aw = jnp.abs(w)
    n, m = aw.shape
    colsum = lax.dot_general(
        jnp.ones((1, n), jnp.float32), aw, (((1,), (0,)), ((), ())),
        precision=lax.Precision.HIGHEST,
    )
    tot = lax.dot_general(
        colsum, jnp.ones((1, m), jnp.float32), (((1,), (1,)), ((), ())),
        precision=lax.Precision.HIGHEST,
    )[0, 0]
    s = 1.0 / jnp.clip(tot / (n * m), 1e-05, None)
    q = jnp.clip(jnp.round(w * s), -1, 1) / s
    return w + (q - w)


def _bf16_dot_t(x, w):
    return lax.dot_general(
        x.astype(jnp.bfloat16),
        w.astype(jnp.bfloat16),
        (((1,), (1,)), ((), ())),
        preferred_element_type=jnp.float32,
    )


def _mlp_body(xg_ref, w1_ref, w2_ref, out_ref):
    xg = xg_ref[...]
    y = _bf16_dot_t(_quant_act(xg), _quant_w(w1_ref[0]))
    g = y[:, :INTER]
    h = y[:, INTER:]
    a = g * lax.logistic(g) * h
    out_ref[...] = _bf16_dot_t(_quant_act(a), _quant_w(w2_ref[0]))


def _expert_mlps(dispatch, gate_proj_w, down_proj_w, interpret=False):
    return pl.pallas_call(
        _mlp_body,
        grid=(E,),
        in_specs=[
            pl.BlockSpec((CAP, HIDDEN), lambda e: (e, 0)),
            pl.BlockSpec((1, 2 * INTER, HIDDEN), lambda e: (e, 0, 0)),
            pl.BlockSpec((1, HIDDEN, INTER), lambda e: (e, 0, 0)),
        ],
        out_specs=pl.BlockSpec((CAP, HIDDEN), lambda e: (e, 0)),
        out_shape=jax.ShapeDtypeStruct((E * CAP, HIDDEN), jnp.float32),
        interpret=interpret,
    )(dispatch, gate_proj_w, down_proj_w)


def _build_combine(interpret=False):
    @functools.partial(
        pl.kernel,
        out_type=jax.ShapeDtypeStruct((T, HIDDEN), jnp.float32),
        mesh=_sc_mesh(),
        scratch_types=[
            pltpu.VMEM((E * T,), jnp.int32),
            pltpu.VMEM((2 * TPW,), jnp.int32),
            pltpu.VMEM((2 * TPW,), jnp.float32),
            pltpu.VMEM((TPW,), jnp.int32),
            pltpu.VMEM((TPW,), jnp.int32),
            pltpu.VMEM((TPW,), jnp.float32),
            pltpu.VMEM((TPW,), jnp.float32),
            pltpu.VMEM((TPW, HIDDEN), jnp.float32),
            pltpu.VMEM((TPW, HIDDEN), jnp.float32),
            pltpu.SemaphoreType.DMA,
        ],
        compiler_params=_SC_PARAMS,
        interpret=interpret,
    )
    def combine(eo_hbm, pos_hbm, ti_hbm, tw_hbm, out_hbm,
                pos_v, ti_v, tw_v, s0_v, s1_v, w0_v, w1_v, r0_v, r1_v, sem):
        base = _wid() * TPW
        pltpu.sync_copy(pos_hbm, pos_v)
        pltpu.sync_copy(ti_hbm.at[pl.ds(2 * base, 2 * TPW)], ti_v)
        pltpu.sync_copy(tw_hbm.at[pl.ds(2 * base, 2 * TPW)], tw_v)

        for ch in range(TPW // L):
            tl = ch * L + lax.iota(jnp.int32, L)
            e0 = plsc.load_gather(ti_v, [2 * tl])
            e1 = plsc.load_gather(ti_v, [2 * tl + 1])
            w0 = plsc.load_gather(tw_v, [2 * tl])
            w1 = plsc.load_gather(tw_v, [2 * tl + 1])
            p0 = plsc.load_gather(pos_v, [e0 * T + base + tl])
            p1 = plsc.load_gather(pos_v, [e1 * T + base + tl])
            s0_v[pl.ds(ch * L, L)] = e0 * CAP + jnp.maximum(p0, 0)
            s1_v[pl.ds(ch * L, L)] = e1 * CAP + jnp.maximum(p1, 0)
            w0_v[pl.ds(ch * L, L)] = jnp.where(p0 >= 0, w0, 0.0)
            w1_v[pl.ds(ch * L, L)] = jnp.where(p1 >= 0, w1, 0.0)

        pltpu.async_copy(eo_hbm.at[s0_v], r0_v, sem).wait()
        pltpu.async_copy(eo_hbm.at[s1_v], r1_v, sem).wait()

        def per_tok(t, carry):
            ws0 = plsc.load_gather(w0_v, [jnp.full((L,), 0, jnp.int32) + t])
            ws1 = plsc.load_gather(w1_v, [jnp.full((L,), 0, jnp.int32) + t])
            for j in range(HIDDEN // L):
                sl = pl.ds(j * L, L)
                r0_v[t, sl] = r0_v[t, sl] * ws0 + r1_v[t, sl] * ws1
            return carry

        lax.fori_loop(0, TPW, per_tok, 0)
        pltpu.sync_copy(r0_v, out_hbm.at[pl.ds(base, TPW)])

    return combine


def _moe(x, gate_norm_w, gate_w, gate_proj_w, down_proj_w, interpret=False):
    B, S, H = x.shape
    x_flat = x.reshape(-1, H)

    top_w, top_i = _router(x_flat, gate_norm_w, gate_w)
    ti_flat = top_i.reshape(-1).astype(jnp.int32)
    tw_flat = top_w.reshape(-1)

    dispatch, pos = _build_compact_gather(interpret)(ti_flat, x_flat)
    eo = _expert_mlps(dispatch, gate_proj_w, down_proj_w, interpret)
    out = _build_combine(interpret)(eo, pos.reshape(-1), ti_flat, tw_flat)
    return out.reshape(B, S, H)


def kernel(x, gate_norm_w, gate_w, gate_proj_w, down_proj_w):
    return _moe(x, gate_norm_w, gate_w, gate_proj_w, down_proj_w)


# speedup vs baseline: 1.4205x; 1.4205x over previous
import functools

import jax
import jax.numpy as jnp
from jax import lax
from jax.experimental import pallas as pl
from jax.experimental.pallas import tpu as pltpu
from jax.experimental.pallas import tpu_sc as plsc

HIDDEN = 768
INTER = 1536
E = 8
CAP = 256
T = 2048
EPS = 1e-06

NC = 2
NS = 16
L = 16
NW = NC * NS
TPW = T // NW


def _sc_mesh():
    return plsc.VectorSubcoreMesh(
        core_axis_name="c", subcore_axis_name="s", num_cores=NC, num_subcores=NS
    )


_SC_PARAMS = pltpu.CompilerParams(needs_layout_passes=False)


def _wid():
    return lax.axis_index("s") * NC + lax.axis_index("c")


def _router(x_flat, gate_norm_w, gate_w):
    var = jnp.mean(jnp.square(x_flat), axis=-1, keepdims=True)
    x_norm = x_flat * lax.rsqrt(var + EPS) * gate_norm_w
    v2 = jnp.mean(jnp.square(x_norm), axis=-1, keepdims=True)
    xr = x_norm * lax.rsqrt(v2 + 1e-08)
    s = 127.0 / jnp.clip(jnp.max(jnp.abs(xr), axis=-1, keepdims=True), 1e-05, None)
    xq = jnp.clip(jnp.round(xr * s), -128, 127) / s
    xq = xr + lax.stop_gradient(xq - xr)
    ws = 1.0 / jnp.clip(jnp.mean(jnp.abs(gate_w)), 1e-05, None)
    wq = jnp.clip(jnp.round(gate_w * ws), -1, 1) / ws
    wq = gate_w + lax.stop_gradient(wq - gate_w)
    logits = xq @ wq.T
    probs = jax.nn.softmax(logits, axis=-1)
    top_w, top_i = jax.lax.top_k(probs, 2)
    return top_w, top_i


def _build_compact_gather(interpret=False):
    @functools.partial(
        pl.kernel,
        out_type=(
            jax.ShapeDtypeStruct((E * CAP, HIDDEN), jnp.float32),
            jax.ShapeDtypeStruct((E, T), jnp.int32),
        ),
        mesh=_sc_mesh(),
        scratch_types=[
            pltpu.VMEM((2 * T,), jnp.int32),
            pltpu.VMEM((CAP,), jnp.int32),
            pltpu.VMEM((T,), jnp.int32),
            pltpu.VMEM((TPW, HIDDEN), jnp.float32),
            pltpu.SemaphoreType.DMA,
        ],
        compiler_params=_SC_PARAMS,
        interpret=interpret,
    )
    def compact_gather(ti_hbm, x_hbm, disp_out, pos_out,
                       ti_v, tok_v, pos_v, rows_v, sem):
        wid = _wid()
        e = wid // (NW // E)
        sub = wid % (NW // E)
        pltpu.sync_copy(ti_hbm, ti_v)

        def zero_body(i, carry):
            tok_v[pl.ds(i * L, L)] = jnp.zeros((L,), jnp.int32)
            return carry

        lax.fori_loop(0, CAP // L, zero_body, 0)

        def chunk(i, running):
            base = i * L
            tid = base + lax.iota(jnp.int32, L)
            i0 = plsc.load_gather(ti_v, [2 * tid])
            i1 = plsc.load_gather(ti_v, [2 * tid + 1])
            sel = (i0 == e) | (i1 == e)
            inc = sel.astype(jnp.int32)
            pos = running + plsc.cumsum(inc) - 1
            keep = sel & (pos < CAP)
            pos_v[pl.ds(base, L)] = jnp.where(keep, pos, -1)
            plsc.store_scatter(tok_v, [jnp.where(keep, pos, 0)], tid, mask=keep)
            return running + jnp.sum(inc)

        lax.fori_loop(0, T // L, chunk, 0)

        @pl.when(sub == 0)
        def _():
            pltpu.sync_copy(pos_v, pos_out.at[e])

        pltpu.async_copy(
            x_hbm.at[tok_v.at[pl.ds(sub * TPW, TPW)]], rows_v, sem
        ).wait()
        pltpu.sync_copy(rows_v, disp_out.at[pl.ds(wid * TPW, TPW)])

    return compact_gather


def _quant_act(x):
    v = jnp.mean(jnp.square(x), axis=-1, keepdims=True)
    xr = x * lax.rsqrt(v + 1e-08)
    s = 127.0 / jnp.clip(jnp.max(jnp.abs(xr), axis=-1, keepdims=True), 1e-05, None)
    q = jnp.clip(jnp.round(xr * s), -128, 127) / s
    return xr + (q - xr)


def _quant_w(w):
    n = w.shape[0] * w.shape[1]
    aw = jnp.sum(jnp.abs(w), axis=0)
    s = 1.0 / jnp.clip(jnp.sum(aw) / n, 1e-05, None)
    c = 1.0 / s
    t = jnp.clip(jnp.round(w * s), -1, 1)
    return t * c


def _bf16_dot_t(x, w):
    return lax.dot_general(
        x.astype(jnp.bfloat16),
        w.astype(jnp.bfloat16),
        (((1,), (1,)), ((), ())),
        preferred_element_type=jnp.float32,
    )


def _mlp_body(xg_ref, w1_ref, w2_ref, out_ref):
    xg = xg_ref[...]
    y = _bf16_dot_t(_quant_act(xg), _quant_w(w1_ref[0]))
    g = y[:, :INTER]
    h = y[:, INTER:]
    a = g * lax.logistic(g) * h
    out_ref[...] = _bf16_dot_t(_quant_act(a), _quant_w(w2_ref[0]))


def _expert_mlps(dispatch, gate_proj_w, down_proj_w, interpret=False):
    return pl.pallas_call(
        _mlp_body,
        grid=(E,),
        in_specs=[
            pl.BlockSpec((CAP, HIDDEN), lambda e: (e, 0)),
            pl.BlockSpec((1, 2 * INTER, HIDDEN), lambda e: (e, 0, 0)),
            pl.BlockSpec((1, HIDDEN, INTER), lambda e: (e, 0, 0)),
        ],
        out_specs=pl.BlockSpec((CAP, HIDDEN), lambda e: (e, 0)),
        out_shape=jax.ShapeDtypeStruct((E * CAP, HIDDEN), jnp.float32),
        interpret=interpret,
    )(dispatch, gate_proj_w, down_proj_w)


def _build_combine(interpret=False):
    @functools.partial(
        pl.kernel,
        out_type=jax.ShapeDtypeStruct((T, HIDDEN), jnp.float32),
        mesh=_sc_mesh(),
        scratch_types=[
            pltpu.VMEM((E * T,), jnp.int32),
            pltpu.VMEM((2 * TPW,), jnp.int32),
            pltpu.VMEM((2 * TPW,), jnp.float32),
            pltpu.VMEM((TPW,), jnp.int32),
            pltpu.VMEM((TPW,), jnp.int32),
            pltpu.VMEM((TPW,), jnp.float32),
            pltpu.VMEM((TPW,), jnp.float32),
            pltpu.VMEM((TPW, HIDDEN), jnp.float32),
            pltpu.VMEM((TPW, HIDDEN), jnp.float32),
            pltpu.SemaphoreType.DMA,
        ],
        compiler_params=_SC_PARAMS,
        interpret=interpret,
    )
    def combine(eo_hbm, pos_hbm, ti_hbm, tw_hbm, out_hbm,
                pos_v, ti_v, tw_v, s0_v, s1_v, w0_v, w1_v, r0_v, r1_v, sem):
        base = _wid() * TPW
        pltpu.sync_copy(pos_hbm, pos_v)
        pltpu.sync_copy(ti_hbm.at[pl.ds(2 * base, 2 * TPW)], ti_v)
        pltpu.sync_copy(tw_hbm.at[pl.ds(2 * base, 2 * TPW)], tw_v)

        for ch in range(TPW // L):
            tl = ch * L + lax.iota(jnp.int32, L)
            e0 = plsc.load_gather(ti_v, [2 * tl])
            e1 = plsc.load_gather(ti_v, [2 * tl + 1])
            w0 = plsc.load_gather(tw_v, [2 * tl])
            w1 = plsc.load_gather(tw_v, [2 * tl + 1])
            p0 = plsc.load_gather(pos_v, [e0 * T + base + tl])
            p1 = plsc.load_gather(pos_v, [e1 * T + base + tl])
            s0_v[pl.ds(ch * L, L)] = e0 * CAP + jnp.maximum(p0, 0)
            s1_v[pl.ds(ch * L, L)] = e1 * CAP + jnp.maximum(p1, 0)
            w0_v[pl.ds(ch * L, L)] = jnp.where(p0 >= 0, w0, 0.0)
            w1_v[pl.ds(ch * L, L)] = jnp.where(p1 >= 0, w1, 0.0)

        pltpu.async_copy(eo_hbm.at[s0_v], r0_v, sem).wait()
        pltpu.async_copy(eo_hbm.at[s1_v], r1_v, sem).wait()

        def per_tok(t, carry):
            ws0 = plsc.load_gather(w0_v, [jnp.full((L,), 0, jnp.int32) + t])
            ws1 = plsc.load_gather(w1_v, [jnp.full((L,), 0, jnp.int32) + t])
            for j in range(HIDDEN // L):
                sl = pl.ds(j * L, L)
                r0_v[t, sl] = r0_v[t, sl] * ws0 + r1_v[t, sl] * ws1
            return carry

        lax.fori_loop(0, TPW, per_tok, 0)
        pltpu.sync_copy(r0_v, out_hbm.at[pl.ds(base, TPW)])

    return combine


def _moe(x, gate_norm_w, gate_w, gate_proj_w, down_proj_w, interpret=False):
    B, S, H = x.shape
    x_flat = x.reshape(-1, H)

    top_w, top_i = _router(x_flat, gate_norm_w, gate_w)
    ti_flat = top_i.reshape(-1).astype(jnp.int32)
    tw_flat = top_w.reshape(-1)

    dispatch, pos = _build_compact_gather(interpret)(ti_flat, x_flat)
    eo = _expert_mlps(dispatch, gate_proj_w, down_proj_w, interpret)
    out = _build_combine(interpret)(eo, pos.reshape(-1), ti_flat, tw_flat)
    return out.reshape(B, S, H)


def kernel(x, gate_norm_w, gate_w, gate_proj_w, down_proj_w):
    return _moe(x, gate_norm_w, gate_w, gate_proj_w, down_proj_w)


# speedup vs baseline: 1.4539x; 1.0235x over previous
import functools

import jax
import jax.numpy as jnp
from jax import lax
from jax.experimental import pallas as pl
from jax.experimental.pallas import tpu as pltpu
from jax.experimental.pallas import tpu_sc as plsc

HIDDEN = 768
INTER = 1536
E = 8
CAP = 256
T = 2048
EPS = 1e-06

NC = 2
NS = 16
L = 16
NW = NC * NS
TPW = T // NW


def _sc_mesh():
    return plsc.VectorSubcoreMesh(
        core_axis_name="c", subcore_axis_name="s", num_cores=NC, num_subcores=NS
    )


_SC_PARAMS = pltpu.CompilerParams(needs_layout_passes=False)


def _wid():
    return lax.axis_index("s") * NC + lax.axis_index("c")


def _router(x_flat, gate_norm_w, gate_w):
    var = jnp.mean(jnp.square(x_flat), axis=-1, keepdims=True)
    x_norm = x_flat * lax.rsqrt(var + EPS) * gate_norm_w
    v2 = jnp.mean(jnp.square(x_norm), axis=-1, keepdims=True)
    xr = x_norm * lax.rsqrt(v2 + 1e-08)
    s = 127.0 / jnp.clip(jnp.max(jnp.abs(xr), axis=-1, keepdims=True), 1e-05, None)
    xq = jnp.clip(jnp.round(xr * s), -128, 127) / s
    xq = xr + lax.stop_gradient(xq - xr)
    ws = 1.0 / jnp.clip(jnp.mean(jnp.abs(gate_w)), 1e-05, None)
    wq = jnp.clip(jnp.round(gate_w * ws), -1, 1) / ws
    wq = gate_w + lax.stop_gradient(wq - gate_w)
    logits = xq @ wq.T
    probs = jax.nn.softmax(logits, axis=-1)
    top_w, top_i = jax.lax.top_k(probs, 2)
    return top_w, top_i


def _build_compact_gather(interpret=False):
    @functools.partial(
        pl.kernel,
        out_type=(
            jax.ShapeDtypeStruct((E * CAP, HIDDEN), jnp.float32),
            jax.ShapeDtypeStruct((E, T), jnp.int32),
        ),
        mesh=_sc_mesh(),
        scratch_types=[
            pltpu.VMEM((2 * T,), jnp.int32),
            pltpu.VMEM((CAP,), jnp.int32),
            pltpu.VMEM((T,), jnp.int32),
            pltpu.VMEM((TPW, HIDDEN), jnp.float32),
            pltpu.SemaphoreType.DMA,
        ],
        compiler_params=_SC_PARAMS,
        interpret=interpret,
    )
    def compact_gather(ti_hbm, x_hbm, disp_out, pos_out,
                       ti_v, tok_v, pos_v, rows_v, sem):
        wid = _wid()
        e = wid // (NW // E)
        sub = wid % (NW // E)
        pltpu.sync_copy(ti_hbm, ti_v)

        def zero_body(i, carry):
            tok_v[pl.ds(i * L, L)] = jnp.zeros((L,), jnp.int32)
            return carry

        lax.fori_loop(0, CAP // L, zero_body, 0)

        def chunk(i, running):
            base = i * L
            tid = base + lax.iota(jnp.int32, L)
            i0 = plsc.load_gather(ti_v, [2 * tid])
            i1 = plsc.load_gather(ti_v, [2 * tid + 1])
            sel = (i0 == e) | (i1 == e)
            inc = sel.astype(jnp.int32)
            pos = running + plsc.cumsum(inc) - 1
            keep = sel & (pos < CAP)
            pos_v[pl.ds(base, L)] = jnp.where(keep, pos, -1)
            plsc.store_scatter(tok_v, [jnp.where(keep, pos, 0)], tid, mask=keep)
            return running + jnp.sum(inc)

        lax.fori_loop(0, T // L, chunk, 0)

        @pl.when(sub == 0)
        def _():
            pltpu.sync_copy(pos_v, pos_out.at[e])

        pltpu.async_copy(
            x_hbm.at[tok_v.at[pl.ds(sub * TPW, TPW)]], rows_v, sem
        ).wait()
        pltpu.sync_copy(rows_v, disp_out.at[pl.ds(wid * TPW, TPW)])

    return compact_gather


def _quant_act(x):
    v = jnp.mean(jnp.square(x), axis=-1, keepdims=True)
    xr = x * lax.rsqrt(v + 1e-08)
    s = 127.0 / jnp.clip(jnp.max(jnp.abs(xr), axis=-1, keepdims=True), 1e-05, None)
    return jnp.clip(jnp.round(xr * s), -128, 127) / s


def _quant_w(w):
    n = w.shape[0] * w.shape[1]
    aw = jnp.sum(jnp.abs(w), axis=0)
    s = 1.0 / jnp.clip(jnp.sum(aw) / n, 1e-05, None)
    c = 1.0 / s
    t = jnp.clip(jnp.round(w * s), -1, 1)
    return t, c


def _bf16_dot_t(x, w):
    return lax.dot_general(
        x.astype(jnp.bfloat16),
        w.astype(jnp.bfloat16),
        (((1,), (1,)), ((), ())),
        preferred_element_type=jnp.float32,
    )


def _mlp_body(xg_ref, w1_ref, w2_ref, out_ref):
    xg = xg_ref[...]
    t1, c1 = _quant_w(w1_ref[0])
    y = _bf16_dot_t(_quant_act(xg), t1) * c1
    g = y[:, :INTER]
    h = y[:, INTER:]
    a = g * lax.logistic(g) * h
    t2, c2 = _quant_w(w2_ref[0])
    out_ref[...] = _bf16_dot_t(_quant_act(a), t2) * c2


def _expert_mlps(dispatch, gate_proj_w, down_proj_w, interpret=False):
    return pl.pallas_call(
        _mlp_body,
        grid=(E,),
        in_specs=[
            pl.BlockSpec((CAP, HIDDEN), lambda e: (e, 0)),
            pl.BlockSpec((1, 2 * INTER, HIDDEN), lambda e: (e, 0, 0)),
            pl.BlockSpec((1, HIDDEN, INTER), lambda e: (e, 0, 0)),
        ],
        out_specs=pl.BlockSpec((CAP, HIDDEN), lambda e: (e, 0)),
        out_shape=jax.ShapeDtypeStruct((E * CAP, HIDDEN), jnp.float32),
        interpret=interpret,
    )(dispatch, gate_proj_w, down_proj_w)


def _build_combine(interpret=False):
    @functools.partial(
        pl.kernel,
        out_type=jax.ShapeDtypeStruct((T, HIDDEN), jnp.float32),
        mesh=_sc_mesh(),
        scratch_types=[
            pltpu.VMEM((E * T,), jnp.int32),
            pltpu.VMEM((2 * TPW,), jnp.int32),
            pltpu.VMEM((2 * TPW,), jnp.float32),
            pltpu.VMEM((TPW,), jnp.int32),
            pltpu.VMEM((TPW,), jnp.int32),
            pltpu.VMEM((TPW,), jnp.float32),
            pltpu.VMEM((TPW,), jnp.float32),
            pltpu.VMEM((TPW, HIDDEN), jnp.float32),
            pltpu.VMEM((TPW, HIDDEN), jnp.float32),
            pltpu.SemaphoreType.DMA,
        ],
        compiler_params=_SC_PARAMS,
        interpret=interpret,
    )
    def combine(eo_hbm, pos_hbm, ti_hbm, tw_hbm, out_hbm,
                pos_v, ti_v, tw_v, s0_v, s1_v, w0_v, w1_v, r0_v, r1_v, sem):
        base = _wid() * TPW
        pltpu.sync_copy(pos_hbm, pos_v)
        pltpu.sync_copy(ti_hbm.at[pl.ds(2 * base, 2 * TPW)], ti_v)
        pltpu.sync_copy(tw_hbm.at[pl.ds(2 * base, 2 * TPW)], tw_v)

        for ch in range(TPW // L):
            tl = ch * L + lax.iota(jnp.int32, L)
            e0 = plsc.load_gather(ti_v, [2 * tl])
            e1 = plsc.load_gather(ti_v, [2 * tl + 1])
            w0 = plsc.load_gather(tw_v, [2 * tl])
            w1 = plsc.load_gather(tw_v, [2 * tl + 1])
            p0 = plsc.load_gather(pos_v, [e0 * T + base + tl])
            p1 = plsc.load_gather(pos_v, [e1 * T + base + tl])
            s0_v[pl.ds(ch * L, L)] = e0 * CAP + jnp.maximum(p0, 0)
            s1_v[pl.ds(ch * L, L)] = e1 * CAP + jnp.maximum(p1, 0)
            w0_v[pl.ds(ch * L, L)] = jnp.where(p0 >= 0, w0, 0.0)
            w1_v[pl.ds(ch * L, L)] = jnp.where(p1 >= 0, w1, 0.0)

        pltpu.async_copy(eo_hbm.at[s0_v], r0_v, sem).wait()
        pltpu.async_copy(eo_hbm.at[s1_v], r1_v, sem).wait()

        def per_tok(t, carry):
            ws0 = plsc.load_gather(w0_v, [jnp.full((L,), 0, jnp.int32) + t])
            ws1 = plsc.load_gather(w1_v, [jnp.full((L,), 0, jnp.int32) + t])
            for j in range(HIDDEN // L):
                sl = pl.ds(j * L, L)
                r0_v[t, sl] = r0_v[t, sl] * ws0 + r1_v[t, sl] * ws1
            return carry

        lax.fori_loop(0, TPW, per_tok, 0)
        pltpu.sync_copy(r0_v, out_hbm.at[pl.ds(base, TPW)])

    return combine


def _moe(x, gate_norm_w, gate_w, gate_proj_w, down_proj_w, interpret=False):
    B, S, H = x.shape
    x_flat = x.reshape(-1, H)

    top_w, top_i = _router(x_flat, gate_norm_w, gate_w)
    ti_flat = top_i.reshape(-1).astype(jnp.int32)
    tw_flat = top_w.reshape(-1)

    dispatch, pos = _build_compact_gather(interpret)(ti_flat, x_flat)
    eo = _expert_mlps(dispatch, gate_proj_w, down_proj_w, interpret)
    out = _build_combine(interpret)(eo, pos.reshape(-1), ti_flat, tw_flat)
    return out.reshape(B, S, H)


def kernel(x, gate_norm_w, gate_w, gate_proj_w, down_proj_w):
    return _moe(x, gate_norm_w, gate_w, gate_proj_w, down_proj_w)
